# baseline (speedup 1.0000x reference)
import functools

import jax
import jax.numpy as jnp
from jax import lax
from jax.experimental import pallas as pl
from jax.experimental.pallas import tpu as pltpu
from jax.experimental.pallas import tpu_sc as plsc

_N = 10000
_E = 320000
_D = 128
_TEMP = 101.0

_NC = 2
_NS = 16
_NW = _NC * _NS
_EPW = _E // _NW
_C = 80
_EPWP = 10080
_EPAD = _NW * _EPWP - _E
_NCHP = _EPWP // _C
_NPH = _NCHP // 2
_NB = _NPH // 2
_NACC = _N + 8
_NPAD = 10240
_SL = _NPAD // _NS
_RB = 624
_RTAIL = _N - _RB * _NS
_R = 400

_mesh = plsc.VectorSubcoreMesh(core_axis_name="c", subcore_axis_name="s")
_sc_params = pltpu.CompilerParams(needs_layout_passes=False)


def _deg_body(dst_hbm, out_hbm, didx, degbuf, cbuf, sumbuf, dshared):
    cid = lax.axis_index("c")
    sid = lax.axis_index("s")
    wid = sid * _NC + cid
    zeros16 = jnp.zeros((16,), jnp.float32)
    ones16 = jnp.ones((16,), jnp.float32)

    def zb(i, c):
        degbuf[pl.ds(i * 16, 16)] = zeros16
        return c
    lax.fori_loop(0, _NPAD // 16, zb, 0)

    pltpu.sync_copy(dst_hbm.at[pl.ds(pl.multiple_of(wid * _EPW, 8), _EPW)], didx)

    def eb(i, c):
        idx = didx[pl.ds(i * 16, 16)]
        plsc.addupdate_scatter(degbuf, [idx], ones16)
        return c
    lax.fori_loop(0, _EPW // 16, eb, 0)

    pltpu.sync_copy(degbuf, dshared.at[pl.ds(pl.multiple_of(sid * _NPAD, 128), _NPAD)])
    plsc.subcore_barrier()
    for t in range(_NS):
        pltpu.sync_copy(
            dshared.at[pl.ds(pl.multiple_of(t * _NPAD + sid * _SL, 128), _SL)],
            cbuf.at[pl.ds(t * _SL, _SL)])

    def sb(j, c):
        acc = cbuf[pl.ds(j * 16, 16)]
        for t in range(1, _NS):
            acc = acc + cbuf[pl.ds(t * _SL + j * 16, 16)]
        sumbuf[pl.ds(j * 16, 16)] = acc
        return c
    lax.fori_loop(0, _SL // 16, sb, 0)

    pltpu.sync_copy(
        sumbuf,
        out_hbm.at[pl.ds(pl.multiple_of(cid * _NPAD + sid * _SL, 128), _SL)])


_deg_call = pl.kernel(
    _deg_body,
    out_type=jax.ShapeDtypeStruct((_NC * _NPAD,), jnp.float32),
    mesh=_mesh,
    scratch_types=[
        pltpu.VMEM((_EPW,), jnp.int32),
        pltpu.VMEM((_NPAD,), jnp.float32),
        pltpu.VMEM((_NS * _SL,), jnp.float32),
        pltpu.VMEM((_SL,), jnp.float32),
        pltpu.VMEM_SHARED((_NS * _NPAD,), jnp.float32),
    ],
    compiler_params=_sc_params,
)


def _scat_body(yn_hbm, src_hbm, dst_hbm, out_hbm, sa0, sa1, sb0, sb1,
               da0, da1, db0, db1, rowsa, rowsb, acc,
               gsema, gsemb, isema, isemb):
    cid = lax.axis_index("c")
    sid = lax.axis_index("s")
    wid = sid * _NC + cid
    r0 = pl.multiple_of(sid * _RB, 8)
    pltpu.sync_copy(yn_hbm.at[pl.ds(r0, _RB)], acc.at[pl.ds(r0, _RB)])

    @pl.when(sid == _NS - 1)
    def _():
        pltpu.sync_copy(yn_hbm.at[pl.ds(_RB * _NS, _RTAIL)],
                        acc.at[pl.ds(_RB * _NS, _RTAIL)])

    plsc.subcore_barrier()
    base = wid * _EPWP

    def fire_idx(p, s0, s1, d0, d1, isem):
        for j, sb, db in ((0, s0, d0), (1, s1, d1)):
            off = pl.multiple_of(base + (2 * p + j) * _C, 8)
            pltpu.async_copy(src_hbm.at[pl.ds(off, _C)], sb, isem)
            pltpu.async_copy(dst_hbm.at[pl.ds(off, _C)], db, isem)

    def drain_idx(s0, s1, d0, d1, isem):
        for b in (s0, s1, d0, d1):
            pltpu.make_async_copy(src_hbm.at[pl.ds(0, _C)], b, isem).wait()

    def fire_g(rows, s0, s1, gsem):
        pltpu.async_copy(yn_hbm.at[s0], rows.at[0], gsem)
        pltpu.async_copy(yn_hbm.at[s1], rows.at[1], gsem)

    def drain_g(rows, gsem):
        for j in range(2):
            pltpu.make_async_copy(yn_hbm.at[pl.ds(0, _C)], rows.at[j], gsem).wait()

    def scat(rows, d0, d1):
        pltpu.sync_copy(rows.at[0], acc.at[d0], add=True)
        pltpu.sync_copy(rows.at[1], acc.at[d1], add=True)

    fire_idx(0, sa0, sa1, da0, da1, isema)
    fire_idx(1, sb0, sb1, db0, db1, isemb)
    drain_idx(sa0, sa1, da0, da1, isema)
    fire_g(rowsa, sa0, sa1, gsema)

    def body(t, c):
        pa = t * 2
        drain_idx(sb0, sb1, db0, db1, isemb)
        fire_g(rowsb, sb0, sb1, gsemb)
        drain_g(rowsa, gsema)
        scat(rowsa, da0, da1)
        fire_idx(pa + 2, sa0, sa1, da0, da1, isema)
        drain_g(rowsb, gsemb)
        scat(rowsb, db0, db1)

        @pl.when(t < _NB - 1)
        def _():
            fire_idx(pa + 3, sb0, sb1, db0, db1, isemb)
        drain_idx(sa0, sa1, da0, da1, isema)
        fire_g(rowsa, sa0, sa1, gsema)
        return c
    lax.fori_loop(0, _NB, body, 0)

    drain_g(rowsa, gsema)
    scat(rowsa, da0, da1)

    plsc.subcore_barrier()
    pltpu.sync_copy(acc.at[pl.ds(r0, _RB)], out_hbm.at[cid, pl.ds(r0, _RB)])

    @pl.when(sid == _NS - 1)
    def _():
        pltpu.sync_copy(acc.at[pl.ds(_RB * _NS, _RTAIL)],
                        out_hbm.at[cid, pl.ds(_RB * _NS, _RTAIL)])


_scat_call = pl.kernel(
    _scat_body,
    out_type=jax.ShapeDtypeStruct((_NC, _N, _D), jnp.float32),
    mesh=_mesh,
    scratch_types=(
        [pltpu.VMEM((_C,), jnp.int32)] * 8
        + [pltpu.VMEM((2, _C, _D), jnp.float32)] * 2
        + [pltpu.VMEM_SHARED((_NACC, _D), jnp.float32)]
        + [pltpu.SemaphoreType.DMA] * 4
    ),
    compiler_params=_sc_params,
)


def _prep_body(deg_ref, x_ref, dis_ref, xn_ref):
    deg = deg_ref[0] + deg_ref[1] + 1.0
    dis = lax.rsqrt(deg)
    dis_ref[...] = dis
    xn_ref[...] = x_ref[...] * dis


_prep_call = pl.pallas_call(
    _prep_body,
    grid=(_N // _R,),
    in_specs=[
        pl.BlockSpec((_NC, _R, 1), lambda i: (0, i, 0)),
        pl.BlockSpec((_R, _D), lambda i: (i, 0)),
    ],
    out_specs=[
        pl.BlockSpec((_R, 1), lambda i: (i, 0)),
        pl.BlockSpec((_R, _D), lambda i: (i, 0)),
    ],
    out_shape=[
        jax.ShapeDtypeStruct((_N, 1), jnp.float32),
        jax.ShapeDtypeStruct((_N, _D), jnp.float32),
    ],
)


def _ct(lhs, rhs):
    return lax.dot_general(lhs, rhs, (((1,), (1,)), ((), ())),
                           preferred_element_type=jnp.float32)


def _gated_experts(dis_ref, yn_ref, s_ref, tf_ref, G_ref, W_ref, b_ref):
    dis = dis_ref[...]
    agg = dis * (s_ref[0] + s_ref[1] - yn_ref[...])
    logits = _ct(tf_ref[...], G_ref[...]) * (1.0 / _TEMP)
    m = jnp.max(logits, axis=-1, keepdims=True)
    e = jnp.exp(logits - m)
    gate = e / jnp.sum(e, axis=-1, keepdims=True)
    h = jnp.zeros_like(agg)
    for i in range(3):
        eo = jnp.maximum(_ct(agg, W_ref[i]) + b_ref[i], 0.0)
        h = h + gate[:, i:i + 1] * eo
    return h, dis


def _layer1_body(dis_ref, yn_ref, s_ref, tf_ref, G_ref, W_ref, b_ref, out_ref):
    h, dis = _gated_experts(dis_ref, yn_ref, s_ref, tf_ref, G_ref, W_ref, b_ref)
    out_ref[...] = h * dis


def _layer2_body(dis_ref, yn_ref, s_ref, tf_ref, G_ref, W_ref, b_ref,
                 fcW_ref, fcb_ref, out_ref):
    h, _ = _gated_experts(dis_ref, yn_ref, s_ref, tf_ref, G_ref, W_ref, b_ref)
    out_ref[...] = _ct(h, fcW_ref[...]) + fcb_ref[...]


_layer_in_specs = [
    pl.BlockSpec((_R, 1), lambda i: (i, 0)),
    pl.BlockSpec((_R, _D), lambda i: (i, 0)),
    pl.BlockSpec((_NC, _R, _D), lambda i: (0, i, 0)),
    pl.BlockSpec((_R, 4), lambda i: (i, 0)),
    pl.BlockSpec((3, 4), lambda i: (0, 0)),
    pl.BlockSpec((3, _D, _D), lambda i: (0, 0, 0)),
    pl.BlockSpec((3, _D), lambda i: (0, 0)),
]

_layer1_call = pl.pallas_call(
    _layer1_body,
    grid=(_N // _R,),
    in_specs=_layer_in_specs,
    out_specs=pl.BlockSpec((_R, _D), lambda i: (i, 0)),
    out_shape=jax.ShapeDtypeStruct((_N, _D), jnp.float32),
)

_layer2_call = pl.pallas_call(
    _layer2_body,
    grid=(_N // _R,),
    in_specs=_layer_in_specs + [
        pl.BlockSpec((_D, _D), lambda i: (0, 0)),
        pl.BlockSpec((1, _D), lambda i: (0, 0)),
    ],
    out_specs=pl.BlockSpec((_R, _D), lambda i: (i, 0)),
    out_shape=jax.ShapeDtypeStruct((_N, _D), jnp.float32),
)


def kernel(x, edge_index, top_features, W1, b1, W2, b2, G1, G2, fcW, fcb):
    src = edge_index[0]
    dst = edge_index[1]
    srcp = jnp.concatenate([src, jnp.zeros((_EPAD,), jnp.int32)])
    dstp = jnp.concatenate([dst, jnp.full((_EPAD,), _N, jnp.int32)])

    deg2 = _deg_call(dst).reshape(_NC, _NPAD)[:, :_N].reshape(_NC, _N, 1)
    dis, xn = _prep_call(deg2, x)

    s1 = _scat_call(xn, srcp, dstp)
    hn = _layer1_call(dis, xn, s1, top_features, G1, W1, b1)

    s2 = _scat_call(hn, srcp, dstp)
    return _layer2_call(dis, hn, s2, top_features, G2, W2, b2,
                        fcW, fcb.reshape(1, _D))


# speedup vs baseline: 1.9154x; 1.9154x over previous
import functools

import jax
import jax.numpy as jnp
from jax import lax
from jax.experimental import pallas as pl
from jax.experimental.pallas import tpu as pltpu
from jax.experimental.pallas import tpu_sc as plsc

_N = 10000
_E = 320000
_D = 128
_TEMP = 101.0

_NC = 2
_NS = 16
_NW = _NC * _NS
_EPW = _E // _NW
_C = 80
_NCH = _EPW // _C
_NB = (_NCH - 1) // 2
_NPAD = 10240
_SL = _NPAD // _NS
_RB = 624
_RTAIL = _N - _RB * _NS
_R = 1000

_mesh = plsc.VectorSubcoreMesh(core_axis_name="c", subcore_axis_name="s")
_sc_params = pltpu.CompilerParams(needs_layout_passes=False)


def _deg_body(dst_hbm, out_hbm, didx, degbuf, cbuf, sumbuf, dshared):
    cid = lax.axis_index("c")
    sid = lax.axis_index("s")
    wid = sid * _NC + cid
    zeros16 = jnp.zeros((16,), jnp.float32)
    ones16 = jnp.ones((16,), jnp.float32)

    def zb(i, c):
        degbuf[pl.ds(i * 16, 16)] = zeros16
        return c
    lax.fori_loop(0, _NPAD // 16, zb, 0)

    pltpu.sync_copy(dst_hbm.at[pl.ds(pl.multiple_of(wid * _EPW, 8), _EPW)], didx)

    def eb(i, c):
        idx = didx[pl.ds(i * 16, 16)]
        plsc.addupdate_scatter(degbuf, [idx], ones16)
        return c
    lax.fori_loop(0, _EPW // 16, eb, 0)

    pltpu.sync_copy(degbuf, dshared.at[pl.ds(pl.multiple_of(sid * _NPAD, 128), _NPAD)])
    plsc.subcore_barrier()
    for t in range(_NS):
        pltpu.sync_copy(
            dshared.at[pl.ds(pl.multiple_of(t * _NPAD + sid * _SL, 128), _SL)],
            cbuf.at[pl.ds(t * _SL, _SL)])

    def sb(j, c):
        acc = cbuf[pl.ds(j * 16, 16)]
        for t in range(1, _NS):
            acc = acc + cbuf[pl.ds(t * _SL + j * 16, 16)]
        sumbuf[pl.ds(j * 16, 16)] = acc
        return c
    lax.fori_loop(0, _SL // 16, sb, 0)

    pltpu.sync_copy(
        sumbuf,
        out_hbm.at[pl.ds(pl.multiple_of(cid * _NPAD + sid * _SL, 128), _SL)])


_deg_call = pl.kernel(
    _deg_body,
    out_type=jax.ShapeDtypeStruct((_NC * _NPAD,), jnp.float32),
    mesh=_mesh,
    scratch_types=[
        pltpu.VMEM((_EPW,), jnp.int32),
        pltpu.VMEM((_NPAD,), jnp.float32),
        pltpu.VMEM((_NS * _SL,), jnp.float32),
        pltpu.VMEM((_SL,), jnp.float32),
        pltpu.VMEM_SHARED((_NS * _NPAD,), jnp.float32),
    ],
    compiler_params=_sc_params,
)


def _scat_body(yn_hbm, src_hbm, dst_hbm, out_hbm, sidx, didxa, didxb,
               rowsa, rowsb, acc, gsema, gsemb, isema, isemb):
    cid = lax.axis_index("c")
    sid = lax.axis_index("s")
    wid = sid * _NC + cid
    r0 = pl.multiple_of(sid * _RB, 8)
    pltpu.sync_copy(yn_hbm.at[pl.ds(r0, _RB)], acc.at[pl.ds(r0, _RB)])

    @pl.when(sid == _NS - 1)
    def _():
        pltpu.sync_copy(yn_hbm.at[pl.ds(_RB * _NS, _RTAIL)],
                        acc.at[pl.ds(_RB * _NS, _RTAIL)])

    pltpu.sync_copy(src_hbm.at[wid], sidx)
    plsc.subcore_barrier()
    base = wid * _EPW

    def fire(g, rows, gsem, didx, isem):
        gd = pltpu.async_copy(yn_hbm.at[sidx.at[g]], rows, gsem)
        off = pl.multiple_of(base + g * _C, 8)
        idd = pltpu.async_copy(dst_hbm.at[pl.ds(off, _C)], didx, isem)
        return gd, idd

    def drain(rows, gsem, didx, isem):
        pltpu.make_async_copy(yn_hbm.at[pl.ds(0, _C)], rows, gsem).wait()
        pltpu.make_async_copy(dst_hbm.at[pl.ds(0, _C)], didx, isem).wait()

    def scat(rows, didx):
        pltpu.sync_copy(rows, acc.at[didx], add=True)

    fire(0, rowsa, gsema, didxa, isema)

    def body(t, c):
        ga = t * 2
        gb, ib = fire(ga + 1, rowsb, gsemb, didxb, isemb)
        drain(rowsa, gsema, didxa, isema)
        scat(rowsa, didxa)
        fire(ga + 2, rowsa, gsema, didxa, isema)
        gb.wait()
        ib.wait()
        scat(rowsb, didxb)
        return c
    lax.fori_loop(0, _NB, body, 0)

    drain(rowsa, gsema, didxa, isema)
    scat(rowsa, didxa)

    plsc.subcore_barrier()
    pltpu.sync_copy(acc.at[pl.ds(r0, _RB)], out_hbm.at[cid, pl.ds(r0, _RB)])

    @pl.when(sid == _NS - 1)
    def _():
        pltpu.sync_copy(acc.at[pl.ds(_RB * _NS, _RTAIL)],
                        out_hbm.at[cid, pl.ds(_RB * _NS, _RTAIL)])


_scat_call = pl.kernel(
    _scat_body,
    out_type=jax.ShapeDtypeStruct((_NC, _N, _D), jnp.float32),
    mesh=_mesh,
    scratch_types=[
        pltpu.VMEM((_NCH, _C), jnp.int32),
        pltpu.VMEM((_C,), jnp.int32),
        pltpu.VMEM((_C,), jnp.int32),
        pltpu.VMEM((_C, _D), jnp.float32),
        pltpu.VMEM((_C, _D), jnp.float32),
        pltpu.VMEM_SHARED((_N, _D), jnp.float32),
        pltpu.SemaphoreType.DMA,
        pltpu.SemaphoreType.DMA,
        pltpu.SemaphoreType.DMA,
        pltpu.SemaphoreType.DMA,
    ],
    compiler_params=_sc_params,
)


def _prep_body(deg_ref, x_ref, dis_ref, xn_ref):
    deg = deg_ref[0] + deg_ref[1] + 1.0
    dis = lax.rsqrt(deg)
    dis_ref[...] = dis
    xn_ref[...] = x_ref[...] * dis


_prep_call = pl.pallas_call(
    _prep_body,
    grid=(_N // _R,),
    in_specs=[
        pl.BlockSpec((_NC, _R, 1), lambda i: (0, i, 0)),
        pl.BlockSpec((_R, _D), lambda i: (i, 0)),
    ],
    out_specs=[
        pl.BlockSpec((_R, 1), lambda i: (i, 0)),
        pl.BlockSpec((_R, _D), lambda i: (i, 0)),
    ],
    out_shape=[
        jax.ShapeDtypeStruct((_N, 1), jnp.float32),
        jax.ShapeDtypeStruct((_N, _D), jnp.float32),
    ],
)


def _ct(lhs, rhs):
    return lax.dot_general(lhs, rhs, (((1,), (1,)), ((), ())),
                           preferred_element_type=jnp.float32)


def _gated_experts(dis_ref, yn_ref, s_ref, tf_ref, G_ref, W_ref, b_ref):
    dis = dis_ref[...]
    agg = dis * (s_ref[0] + s_ref[1] - yn_ref[...])
    logits = _ct(tf_ref[...], G_ref[...]) * (1.0 / _TEMP)
    m = jnp.max(logits, axis=-1, keepdims=True)
    e = jnp.exp(logits - m)
    gate = e / jnp.sum(e, axis=-1, keepdims=True)
    h = jnp.zeros_like(agg)
    for i in range(3):
        eo = jnp.maximum(_ct(agg, W_ref[i]) + b_ref[i], 0.0)
        h = h + gate[:, i:i + 1] * eo
    return h, dis


def _layer1_body(dis_ref, yn_ref, s_ref, tf_ref, G_ref, W_ref, b_ref, out_ref):
    h, dis = _gated_experts(dis_ref, yn_ref, s_ref, tf_ref, G_ref, W_ref, b_ref)
    out_ref[...] = h * dis


def _layer2_body(dis_ref, yn_ref, s_ref, tf_ref, G_ref, W_ref, b_ref,
                 fcW_ref, fcb_ref, out_ref):
    h, _ = _gated_experts(dis_ref, yn_ref, s_ref, tf_ref, G_ref, W_ref, b_ref)
    out_ref[...] = _ct(h, fcW_ref[...]) + fcb_ref[...]


_layer_in_specs = [
    pl.BlockSpec((_R, 1), lambda i: (i, 0)),
    pl.BlockSpec((_R, _D), lambda i: (i, 0)),
    pl.BlockSpec((_NC, _R, _D), lambda i: (0, i, 0)),
    pl.BlockSpec((_R, 4), lambda i: (i, 0)),
    pl.BlockSpec((3, 4), lambda i: (0, 0)),
    pl.BlockSpec((3, _D, _D), lambda i: (0, 0, 0)),
    pl.BlockSpec((3, _D), lambda i: (0, 0)),
]

_layer1_call = pl.pallas_call(
    _layer1_body,
    grid=(_N // _R,),
    in_specs=_layer_in_specs,
    out_specs=pl.BlockSpec((_R, _D), lambda i: (i, 0)),
    out_shape=jax.ShapeDtypeStruct((_N, _D), jnp.float32),
)

_layer2_call = pl.pallas_call(
    _layer2_body,
    grid=(_N // _R,),
    in_specs=_layer_in_specs + [
        pl.BlockSpec((_D, _D), lambda i: (0, 0)),
        pl.BlockSpec((1, _D), lambda i: (0, 0)),
    ],
    out_specs=pl.BlockSpec((_R, _D), lambda i: (i, 0)),
    out_shape=jax.ShapeDtypeStruct((_N, _D), jnp.float32),
)


def kernel(x, edge_index, top_features, W1, b1, W2, b2, G1, G2, fcW, fcb):
    src = edge_index[0]
    dst = edge_index[1]
    src3 = src.reshape(_NW, _NCH, _C)

    deg2 = _deg_call(dst).reshape(_NC, _NPAD)[:, :_N].reshape(_NC, _N, 1)
    dis, xn = _prep_call(deg2, x)

    s1 = _scat_call(xn, src3, dst)
    hn = _layer1_call(dis, xn, s1, top_features, G1, W1, b1)

    s2 = _scat_call(hn, src3, dst)
    return _layer2_call(dis, hn, s2, top_features, G2, W2, b2,
                        fcW, fcb.reshape(1, _D))


# speedup vs baseline: 2.2098x; 1.1537x over previous
import functools

import jax
import jax.numpy as jnp
from jax import lax
from jax.experimental import pallas as pl
from jax.experimental.pallas import tpu as pltpu
from jax.experimental.pallas import tpu_sc as plsc

_N = 10000
_E = 320000
_D = 128
_TEMP = 101.0

_NC = 2
_NS = 16
_NW = _NC * _NS
_EPW = _E // _NW
_C = 80
_NCH = _EPW // _C
_NB = (_NCH - 2) // 3
_NPAD = 10240
_SL = _NPAD // _NS
_RB = 624
_RTAIL = _N - _RB * _NS
_R = 1000

_mesh = plsc.VectorSubcoreMesh(core_axis_name="c", subcore_axis_name="s")
_sc_params = pltpu.CompilerParams(needs_layout_passes=False)


def _deg_body(dst_hbm, out_hbm, didx, degbuf, cbuf, sumbuf, dshared):
    cid = lax.axis_index("c")
    sid = lax.axis_index("s")
    wid = sid * _NC + cid
    zeros16 = jnp.zeros((16,), jnp.float32)
    ones16 = jnp.ones((16,), jnp.float32)

    def zb(i, c):
        degbuf[pl.ds(i * 16, 16)] = zeros16
        return c
    lax.fori_loop(0, _NPAD // 16, zb, 0)

    pltpu.sync_copy(dst_hbm.at[pl.ds(pl.multiple_of(wid * _EPW, 8), _EPW)], didx)

    def eb(i, c):
        idx = didx[pl.ds(i * 16, 16)]
        plsc.addupdate_scatter(degbuf, [idx], ones16)
        return c
    lax.fori_loop(0, _EPW // 16, eb, 0)

    pltpu.sync_copy(degbuf, dshared.at[pl.ds(pl.multiple_of(sid * _NPAD, 128), _NPAD)])
    plsc.subcore_barrier()
    for t in range(_NS):
        pltpu.sync_copy(
            dshared.at[pl.ds(pl.multiple_of(t * _NPAD + sid * _SL, 128), _SL)],
            cbuf.at[pl.ds(t * _SL, _SL)])

    def sb(j, c):
        acc = cbuf[pl.ds(j * 16, 16)]
        for t in range(1, _NS):
            acc = acc + cbuf[pl.ds(t * _SL + j * 16, 16)]
        sumbuf[pl.ds(j * 16, 16)] = acc
        return c
    lax.fori_loop(0, _SL // 16, sb, 0)

    pltpu.sync_copy(
        sumbuf,
        out_hbm.at[pl.ds(pl.multiple_of(cid * _NPAD + sid * _SL, 128), _SL)])


_deg_call = pl.kernel(
    _deg_body,
    out_type=jax.ShapeDtypeStruct((_NC * _NPAD,), jnp.float32),
    mesh=_mesh,
    scratch_types=[
        pltpu.VMEM((_EPW,), jnp.int32),
        pltpu.VMEM((_NPAD,), jnp.float32),
        pltpu.VMEM((_NS * _SL,), jnp.float32),
        pltpu.VMEM((_SL,), jnp.float32),
        pltpu.VMEM_SHARED((_NS * _NPAD,), jnp.float32),
    ],
    compiler_params=_sc_params,
)


def _scat_body(yn_hbm, src_hbm, dst_hbm, out_hbm, sidx, didxa, didxb, didxc,
               rowsa, rowsb, rowsc, acc, gsema, gsemb, gsemc,
               isema, isemb, isemc):
    cid = lax.axis_index("c")
    sid = lax.axis_index("s")
    wid = sid * _NC + cid
    r0 = pl.multiple_of(sid * _RB, 8)
    pltpu.sync_copy(yn_hbm.at[pl.ds(r0, _RB)], acc.at[pl.ds(r0, _RB)])

    @pl.when(sid == _NS - 1)
    def _():
        pltpu.sync_copy(yn_hbm.at[pl.ds(_RB * _NS, _RTAIL)],
                        acc.at[pl.ds(_RB * _NS, _RTAIL)])

    pltpu.sync_copy(src_hbm.at[wid], sidx)
    plsc.subcore_barrier()
    base = wid * _EPW

    def fire(g, rows, gsem, didx, isem):
        gd = pltpu.async_copy(yn_hbm.at[sidx.at[g]], rows, gsem)
        off = pl.multiple_of(base + g * _C, 8)
        idd = pltpu.async_copy(dst_hbm.at[pl.ds(off, _C)], didx, isem)
        return gd, idd

    def drain(rows, gsem, didx, isem):
        pltpu.make_async_copy(yn_hbm.at[pl.ds(0, _C)], rows, gsem).wait()
        pltpu.make_async_copy(dst_hbm.at[pl.ds(0, _C)], didx, isem).wait()

    def scat(rows, didx):
        pltpu.sync_copy(rows, acc.at[didx], add=True)

    fire(0, rowsa, gsema, didxa, isema)
    fire(1, rowsb, gsemb, didxb, isemb)
    fire(2, rowsc, gsemc, didxc, isemc)

    def body(t, c):
        g = t * 3
        drain(rowsa, gsema, didxa, isema)
        scat(rowsa, didxa)
        fire(g + 3, rowsa, gsema, didxa, isema)
        drain(rowsb, gsemb, didxb, isemb)
        scat(rowsb, didxb)
        fire(g + 4, rowsb, gsemb, didxb, isemb)
        drain(rowsc, gsemc, didxc, isemc)
        scat(rowsc, didxc)

        @pl.when(t < _NB - 1)
        def _():
            fire(g + 5, rowsc, gsemc, didxc, isemc)
        return c
    lax.fori_loop(0, _NB, body, 0)

    drain(rowsa, gsema, didxa, isema)
    scat(rowsa, didxa)
    drain(rowsb, gsemb, didxb, isemb)
    scat(rowsb, didxb)

    plsc.subcore_barrier()
    pltpu.sync_copy(acc.at[pl.ds(r0, _RB)], out_hbm.at[cid, pl.ds(r0, _RB)])

    @pl.when(sid == _NS - 1)
    def _():
        pltpu.sync_copy(acc.at[pl.ds(_RB * _NS, _RTAIL)],
                        out_hbm.at[cid, pl.ds(_RB * _NS, _RTAIL)])


_scat_call = pl.kernel(
    _scat_body,
    out_type=jax.ShapeDtypeStruct((_NC, _N, _D), jnp.float32),
    mesh=_mesh,
    scratch_types=(
        [pltpu.VMEM((_NCH, _C), jnp.int32)]
        + [pltpu.VMEM((_C,), jnp.int32)] * 3
        + [pltpu.VMEM((_C, _D), jnp.float32)] * 3
        + [pltpu.VMEM_SHARED((_N, _D), jnp.float32)]
        + [pltpu.SemaphoreType.DMA] * 6
    ),
    compiler_params=_sc_params,
)


def _prep_body(deg_ref, x_ref, dis_ref, xn_ref):
    deg = deg_ref[0] + deg_ref[1] + 1.0
    dis = lax.rsqrt(deg)
    dis_ref[...] = dis
    xn_ref[...] = x_ref[...] * dis


_prep_call = pl.pallas_call(
    _prep_body,
    grid=(_N // _R,),
    in_specs=[
        pl.BlockSpec((_NC, _R, 1), lambda i: (0, i, 0)),
        pl.BlockSpec((_R, _D), lambda i: (i, 0)),
    ],
    out_specs=[
        pl.BlockSpec((_R, 1), lambda i: (i, 0)),
        pl.BlockSpec((_R, _D), lambda i: (i, 0)),
    ],
    out_shape=[
        jax.ShapeDtypeStruct((_N, 1), jnp.float32),
        jax.ShapeDtypeStruct((_N, _D), jnp.float32),
    ],
)


def _ct(lhs, rhs):
    return lax.dot_general(lhs, rhs, (((1,), (1,)), ((), ())),
                           preferred_element_type=jnp.float32)


def _gated_experts(dis_ref, yn_ref, s_ref, tf_ref, G_ref, W_ref, b_ref):
    dis = dis_ref[...]
    agg = dis * (s_ref[0] + s_ref[1] - yn_ref[...])
    logits = _ct(tf_ref[...], G_ref[...]) * (1.0 / _TEMP)
    m = jnp.max(logits, axis=-1, keepdims=True)
    e = jnp.exp(logits - m)
    gate = e / jnp.sum(e, axis=-1, keepdims=True)
    h = jnp.zeros_like(agg)
    for i in range(3):
        eo = jnp.maximum(_ct(agg, W_ref[i]) + b_ref[i], 0.0)
        h = h + gate[:, i:i + 1] * eo
    return h, dis


def _layer1_body(dis_ref, yn_ref, s_ref, tf_ref, G_ref, W_ref, b_ref, out_ref):
    h, dis = _gated_experts(dis_ref, yn_ref, s_ref, tf_ref, G_ref, W_ref, b_ref)
    out_ref[...] = h * dis


def _layer2_body(dis_ref, yn_ref, s_ref, tf_ref, G_ref, W_ref, b_ref,
                 fcW_ref, fcb_ref, out_ref):
    h, _ = _gated_experts(dis_ref, yn_ref, s_ref, tf_ref, G_ref, W_ref, b_ref)
    out_ref[...] = _ct(h, fcW_ref[...]) + fcb_ref[...]


_layer_in_specs = [
    pl.BlockSpec((_R, 1), lambda i: (i, 0)),
    pl.BlockSpec((_R, _D), lambda i: (i, 0)),
    pl.BlockSpec((_NC, _R, _D), lambda i: (0, i, 0)),
    pl.BlockSpec((_R, 4), lambda i: (i, 0)),
    pl.BlockSpec((3, 4), lambda i: (0, 0)),
    pl.BlockSpec((3, _D, _D), lambda i: (0, 0, 0)),
    pl.BlockSpec((3, _D), lambda i: (0, 0)),
]

_layer1_call = pl.pallas_call(
    _layer1_body,
    grid=(_N // _R,),
    in_specs=_layer_in_specs,
    out_specs=pl.BlockSpec((_R, _D), lambda i: (i, 0)),
    out_shape=jax.ShapeDtypeStruct((_N, _D), jnp.float32),
)

_layer2_call = pl.pallas_call(
    _layer2_body,
    grid=(_N // _R,),
    in_specs=_layer_in_specs + [
        pl.BlockSpec((_D, _D), lambda i: (0, 0)),
        pl.BlockSpec((1, _D), lambda i: (0, 0)),
    ],
    out_specs=pl.BlockSpec((_R, _D), lambda i: (i, 0)),
    out_shape=jax.ShapeDtypeStruct((_N, _D), jnp.float32),
)


def kernel(x, edge_index, top_features, W1, b1, W2, b2, G1, G2, fcW, fcb):
    src = edge_index[0]
    dst = edge_index[1]
    src3 = src.reshape(_NW, _NCH, _C)

    deg2 = _deg_call(dst).reshape(_NC, _NPAD)[:, :_N].reshape(_NC, _N, 1)
    dis, xn = _prep_call(deg2, x)

    s1 = _scat_call(xn, src3, dst)
    hn = _layer1_call(dis, xn, s1, top_features, G1, W1, b1)

    s2 = _scat_call(hn, src3, dst)
    return _layer2_call(dis, hn, s2, top_features, G2, W2, b2,
                        fcW, fcb.reshape(1, _D))


# speedup vs baseline: 2.2733x; 1.0287x over previous
import jax
import jax.numpy as jnp
from jax import lax
from jax.experimental import pallas as pl
from jax.experimental.pallas import tpu as pltpu
from jax.experimental.pallas import tpu_sc as plsc

_N = 10000
_E = 320000
_D = 128
_TEMP = 101.0

_NC = 2
_NS = 16
_NW = _NC * _NS
_EPW = _E // _NW
_C = 80
_NCH = _EPW // _C
_NB = (_NCH - 2) // 3
_NPAD = 10240
_SL = _NPAD // _NS
_RB = 624
_RTAIL = _N - _RB * _NS
_R = 2000

_mesh = plsc.VectorSubcoreMesh(core_axis_name="c", subcore_axis_name="s")
_sc_params = pltpu.CompilerParams(needs_layout_passes=False)


def _deg_body(dst_hbm, out_hbm, didx, degbuf, cbuf, sumbuf, dshared):
    cid = lax.axis_index("c")
    sid = lax.axis_index("s")
    wid = sid * _NC + cid
    zeros16 = jnp.zeros((16,), jnp.float32)
    ones16 = jnp.ones((16,), jnp.float32)

    def zb(i, c):
        degbuf[pl.ds(i * 16, 16)] = zeros16
        return c
    lax.fori_loop(0, _NPAD // 16, zb, 0)

    pltpu.sync_copy(dst_hbm.at[pl.ds(pl.multiple_of(wid * _EPW, 8), _EPW)], didx)

    def eb(i, c):
        idx = didx[pl.ds(i * 16, 16)]
        plsc.addupdate_scatter(degbuf, [idx], ones16)
        return c
    lax.fori_loop(0, _EPW // 16, eb, 0)

    pltpu.sync_copy(degbuf, dshared.at[pl.ds(pl.multiple_of(sid * _NPAD, 128), _NPAD)])
    plsc.subcore_barrier()
    for t in range(_NS):
        pltpu.sync_copy(
            dshared.at[pl.ds(pl.multiple_of(t * _NPAD + sid * _SL, 128), _SL)],
            cbuf.at[pl.ds(t * _SL, _SL)])

    def sb(j, c):
        acc = cbuf[pl.ds(j * 16, 16)]
        for t in range(1, _NS):
            acc = acc + cbuf[pl.ds(t * _SL + j * 16, 16)]
        sumbuf[pl.ds(j * 16, 16)] = acc
        return c
    lax.fori_loop(0, _SL // 16, sb, 0)

    pltpu.sync_copy(
        sumbuf,
        out_hbm.at[pl.ds(pl.multiple_of(cid * _NPAD + sid * _SL, 128), _SL)])


_deg_call = pl.kernel(
    _deg_body,
    out_type=jax.ShapeDtypeStruct((_NC * _NPAD,), jnp.float32),
    mesh=_mesh,
    scratch_types=[
        pltpu.VMEM((_EPW,), jnp.int32),
        pltpu.VMEM((_NPAD,), jnp.float32),
        pltpu.VMEM((_NS * _SL,), jnp.float32),
        pltpu.VMEM((_SL,), jnp.float32),
        pltpu.VMEM_SHARED((_NS * _NPAD,), jnp.float32),
    ],
    compiler_params=_sc_params,
)


def _scat_body(yn_hbm, src_hbm, dst_hbm, out_hbm, sidx, didxa, didxb, didxc,
               rowsa, rowsb, rowsc, acc, gsema, gsemb, gsemc,
               isema, isemb, isemc):
    cid = lax.axis_index("c")
    sid = lax.axis_index("s")
    wid = sid * _NC + cid
    r0 = pl.multiple_of(sid * _RB, 8)
    pltpu.sync_copy(yn_hbm.at[pl.ds(r0, _RB)], acc.at[pl.ds(r0, _RB)])

    @pl.when(sid == _NS - 1)
    def _():
        pltpu.sync_copy(yn_hbm.at[pl.ds(_RB * _NS, _RTAIL)],
                        acc.at[pl.ds(_RB * _NS, _RTAIL)])

    pltpu.sync_copy(src_hbm.at[wid], sidx)
    plsc.subcore_barrier()
    base = wid * _EPW

    def fire(g, rows, gsem, didx, isem):
        gd = pltpu.async_copy(yn_hbm.at[sidx.at[g]], rows, gsem)
        off = pl.multiple_of(base + g * _C, 8)
        idd = pltpu.async_copy(dst_hbm.at[pl.ds(off, _C)], didx, isem)
        return gd, idd

    def drain(rows, gsem, didx, isem):
        pltpu.make_async_copy(yn_hbm.at[pl.ds(0, _C)], rows, gsem).wait()
        pltpu.make_async_copy(dst_hbm.at[pl.ds(0, _C)], didx, isem).wait()

    def scat(rows, didx):
        pltpu.sync_copy(rows, acc.at[didx], add=True)

    fire(0, rowsa, gsema, didxa, isema)
    fire(1, rowsb, gsemb, didxb, isemb)
    fire(2, rowsc, gsemc, didxc, isemc)

    def body(t, c):
        g = t * 3
        drain(rowsa, gsema, didxa, isema)
        scat(rowsa, didxa)
        fire(g + 3, rowsa, gsema, didxa, isema)
        drain(rowsb, gsemb, didxb, isemb)
        scat(rowsb, didxb)
        fire(g + 4, rowsb, gsemb, didxb, isemb)
        drain(rowsc, gsemc, didxc, isemc)
        scat(rowsc, didxc)

        @pl.when(t < _NB - 1)
        def _():
            fire(g + 5, rowsc, gsemc, didxc, isemc)
        return c
    lax.fori_loop(0, _NB, body, 0)

    drain(rowsa, gsema, didxa, isema)
    scat(rowsa, didxa)
    drain(rowsb, gsemb, didxb, isemb)
    scat(rowsb, didxb)

    plsc.subcore_barrier()
    pltpu.sync_copy(acc.at[pl.ds(r0, _RB)], out_hbm.at[cid, pl.ds(r0, _RB)])

    @pl.when(sid == _NS - 1)
    def _():
        pltpu.sync_copy(acc.at[pl.ds(_RB * _NS, _RTAIL)],
                        out_hbm.at[cid, pl.ds(_RB * _NS, _RTAIL)])


_scat_call = pl.kernel(
    _scat_body,
    out_type=jax.ShapeDtypeStruct((_NC, _N, _D), jnp.float32),
    mesh=_mesh,
    scratch_types=(
        [pltpu.VMEM((_NCH, _C), jnp.int32)]
        + [pltpu.VMEM((_C,), jnp.int32)] * 3
        + [pltpu.VMEM((_C, _D), jnp.float32)] * 3
        + [pltpu.VMEM_SHARED((_N, _D), jnp.float32)]
        + [pltpu.SemaphoreType.DMA] * 6
    ),
    compiler_params=_sc_params,
)


def _prep_body(deg_ref, x_ref, dis_ref, xn_ref):
    deg = deg_ref[0] + deg_ref[1] + 1.0
    dis = lax.rsqrt(deg)
    dis_ref[...] = dis
    xn_ref[...] = x_ref[...] * dis


_prep_call = pl.pallas_call(
    _prep_body,
    grid=(_N // _R,),
    in_specs=[
        pl.BlockSpec((_NC, _R, 1), lambda i: (0, i, 0)),
        pl.BlockSpec((_R, _D), lambda i: (i, 0)),
    ],
    out_specs=[
        pl.BlockSpec((_R, 1), lambda i: (i, 0)),
        pl.BlockSpec((_R, _D), lambda i: (i, 0)),
    ],
    out_shape=[
        jax.ShapeDtypeStruct((_N, 1), jnp.float32),
        jax.ShapeDtypeStruct((_N, _D), jnp.float32),
    ],
)


def _ct(lhs, rhs):
    return lax.dot_general(lhs, rhs, (((1,), (1,)), ((), ())),
                           preferred_element_type=jnp.float32)


def _gated_experts(dis_ref, yn_ref, s_ref, tf_ref, G_ref, W_ref, b_ref):
    dis = dis_ref[...]
    agg = dis * (s_ref[0] + s_ref[1] - yn_ref[...])
    logits = _ct(tf_ref[...], G_ref[...]) * (1.0 / _TEMP)
    m = jnp.max(logits, axis=-1, keepdims=True)
    e = jnp.exp(logits - m)
    gate = e / jnp.sum(e, axis=-1, keepdims=True)
    h = jnp.zeros_like(agg)
    for i in range(3):
        eo = jnp.maximum(_ct(agg, W_ref[i]) + b_ref[i], 0.0)
        h = h + gate[:, i:i + 1] * eo
    return h, dis


def _layer1_body(dis_ref, yn_ref, s_ref, tf_ref, G_ref, W_ref, b_ref, out_ref):
    h, dis = _gated_experts(dis_ref, yn_ref, s_ref, tf_ref, G_ref, W_ref, b_ref)
    out_ref[...] = h * dis


def _layer2_body(dis_ref, yn_ref, s_ref, tf_ref, G_ref, W_ref, b_ref,
                 fcW_ref, fcb_ref, out_ref):
    h, _ = _gated_experts(dis_ref, yn_ref, s_ref, tf_ref, G_ref, W_ref, b_ref)
    out_ref[...] = _ct(h, fcW_ref[...]) + fcb_ref[...]


_layer_in_specs = [
    pl.BlockSpec((_R, 1), lambda i: (i, 0)),
    pl.BlockSpec((_R, _D), lambda i: (i, 0)),
    pl.BlockSpec((_NC, _R, _D), lambda i: (0, i, 0)),
    pl.BlockSpec((_R, 4), lambda i: (i, 0)),
    pl.BlockSpec((3, 4), lambda i: (0, 0)),
    pl.BlockSpec((3, _D, _D), lambda i: (0, 0, 0)),
    pl.BlockSpec((3, _D), lambda i: (0, 0)),
]

_layer1_call = pl.pallas_call(
    _layer1_body,
    grid=(_N // _R,),
    in_specs=_layer_in_specs,
    out_specs=pl.BlockSpec((_R, _D), lambda i: (i, 0)),
    out_shape=jax.ShapeDtypeStruct((_N, _D), jnp.float32),
)

_layer2_call = pl.pallas_call(
    _layer2_body,
    grid=(_N // _R,),
    in_specs=_layer_in_specs + [
        pl.BlockSpec((_D, _D), lambda i: (0, 0)),
        pl.BlockSpec((1, _D), lambda i: (0, 0)),
    ],
    out_specs=pl.BlockSpec((_R, _D), lambda i: (i, 0)),
    out_shape=jax.ShapeDtypeStruct((_N, _D), jnp.float32),
)


def kernel(x, edge_index, top_features, W1, b1, W2, b2, G1, G2, fcW, fcb):
    src = edge_index[0]
    dst = edge_index[1]
    src3 = src.reshape(_NW, _NCH, _C)

    deg2 = _deg_call(dst).reshape(_NC, _NPAD)[:, :_N].reshape(_NC, _N, 1)
    dis, xn = _prep_call(deg2, x)

    s1 = _scat_call(xn, src3, dst)
    hn = _layer1_call(dis, xn, s1, top_features, G1, W1, b1)

    s2 = _scat_call(hn, src3, dst)
    return _layer2_call(dis, hn, s2, top_features, G2, W2, b2,
                        fcW, fcb.reshape(1, _D))


# speedup vs baseline: 2.2749x; 1.0007x over previous
import jax
import jax.numpy as jnp
from jax import lax
from jax.experimental import pallas as pl
from jax.experimental.pallas import tpu as pltpu
from jax.experimental.pallas import tpu_sc as plsc

_N = 10000
_E = 320000
_D = 128
_TEMP = 101.0

_NC = 2
_NS = 16
_NW = _NC * _NS
_EPW = _E // _NW
_C = 80
_NCH = _EPW // _C
_NB = (_NCH - 2) // 3
_NPAD = 10240
_SL = _NPAD // _NS
_RB = 624
_RTAIL = _N - _RB * _NS
_R = 5000

_mesh = plsc.VectorSubcoreMesh(core_axis_name="c", subcore_axis_name="s")
_sc_params = pltpu.CompilerParams(needs_layout_passes=False)


def _deg_body(dst_hbm, out_hbm, didx, degbuf, cbuf, sumbuf, dshared):
    cid = lax.axis_index("c")
    sid = lax.axis_index("s")
    wid = sid * _NC + cid
    zeros16 = jnp.zeros((16,), jnp.float32)
    ones16 = jnp.ones((16,), jnp.float32)

    def zb(i, c):
        degbuf[pl.ds(i * 16, 16)] = zeros16
        return c
    lax.fori_loop(0, _NPAD // 16, zb, 0)

    pltpu.sync_copy(dst_hbm.at[pl.ds(pl.multiple_of(wid * _EPW, 8), _EPW)], didx)

    def eb(i, c):
        idx = didx[pl.ds(i * 16, 16)]
        plsc.addupdate_scatter(degbuf, [idx], ones16)
        return c
    lax.fori_loop(0, _EPW // 16, eb, 0)

    pltpu.sync_copy(degbuf, dshared.at[pl.ds(pl.multiple_of(sid * _NPAD, 128), _NPAD)])
    plsc.subcore_barrier()
    for t in range(_NS):
        pltpu.sync_copy(
            dshared.at[pl.ds(pl.multiple_of(t * _NPAD + sid * _SL, 128), _SL)],
            cbuf.at[pl.ds(t * _SL, _SL)])

    def sb(j, c):
        acc = cbuf[pl.ds(j * 16, 16)]
        for t in range(1, _NS):
            acc = acc + cbuf[pl.ds(t * _SL + j * 16, 16)]
        sumbuf[pl.ds(j * 16, 16)] = acc
        return c
    lax.fori_loop(0, _SL // 16, sb, 0)

    pltpu.sync_copy(
        sumbuf,
        out_hbm.at[pl.ds(pl.multiple_of(cid * _NPAD + sid * _SL, 128), _SL)])


_deg_call = pl.kernel(
    _deg_body,
    out_type=jax.ShapeDtypeStruct((_NC * _NPAD,), jnp.float32),
    mesh=_mesh,
    scratch_types=[
        pltpu.VMEM((_EPW,), jnp.int32),
        pltpu.VMEM((_NPAD,), jnp.float32),
        pltpu.VMEM((_NS * _SL,), jnp.float32),
        pltpu.VMEM((_SL,), jnp.float32),
        pltpu.VMEM_SHARED((_NS * _NPAD,), jnp.float32),
    ],
    compiler_params=_sc_params,
)


def _scat_body(yn_hbm, src_hbm, dst_hbm, out_hbm, sidx, didxa, didxb, didxc,
               rowsa, rowsb, rowsc, acc, gsema, gsemb, gsemc,
               isema, isemb, isemc):
    cid = lax.axis_index("c")
    sid = lax.axis_index("s")
    wid = sid * _NC + cid
    r0 = pl.multiple_of(sid * _RB, 8)
    pltpu.sync_copy(yn_hbm.at[pl.ds(r0, _RB)], acc.at[pl.ds(r0, _RB)])

    @pl.when(sid == _NS - 1)
    def _():
        pltpu.sync_copy(yn_hbm.at[pl.ds(_RB * _NS, _RTAIL)],
                        acc.at[pl.ds(_RB * _NS, _RTAIL)])

    pltpu.sync_copy(src_hbm.at[wid], sidx)
    plsc.subcore_barrier()
    base = wid * _EPW

    def fire(g, rows, gsem, didx, isem):
        gd = pltpu.async_copy(yn_hbm.at[sidx.at[g]], rows, gsem)
        off = pl.multiple_of(base + g * _C, 8)
        idd = pltpu.async_copy(dst_hbm.at[pl.ds(off, _C)], didx, isem)
        return gd, idd

    def drain(rows, gsem, didx, isem):
        pltpu.make_async_copy(yn_hbm.at[pl.ds(0, _C)], rows, gsem).wait()
        pltpu.make_async_copy(dst_hbm.at[pl.ds(0, _C)], didx, isem).wait()

    def scat(rows, didx):
        pltpu.sync_copy(rows, acc.at[didx], add=True)

    fire(0, rowsa, gsema, didxa, isema)
    fire(1, rowsb, gsemb, didxb, isemb)
    fire(2, rowsc, gsemc, didxc, isemc)

    def body(t, c):
        g = t * 3
        drain(rowsa, gsema, didxa, isema)
        scat(rowsa, didxa)
        fire(g + 3, rowsa, gsema, didxa, isema)
        drain(rowsb, gsemb, didxb, isemb)
        scat(rowsb, didxb)
        fire(g + 4, rowsb, gsemb, didxb, isemb)
        drain(rowsc, gsemc, didxc, isemc)
        scat(rowsc, didxc)

        @pl.when(t < _NB - 1)
        def _():
            fire(g + 5, rowsc, gsemc, didxc, isemc)
        return c
    lax.fori_loop(0, _NB, body, 0)

    drain(rowsa, gsema, didxa, isema)
    scat(rowsa, didxa)
    drain(rowsb, gsemb, didxb, isemb)
    scat(rowsb, didxb)

    plsc.subcore_barrier()
    pltpu.sync_copy(acc.at[pl.ds(r0, _RB)], out_hbm.at[cid, pl.ds(r0, _RB)])

    @pl.when(sid == _NS - 1)
    def _():
        pltpu.sync_copy(acc.at[pl.ds(_RB * _NS, _RTAIL)],
                        out_hbm.at[cid, pl.ds(_RB * _NS, _RTAIL)])


_scat_call = pl.kernel(
    _scat_body,
    out_type=jax.ShapeDtypeStruct((_NC, _N, _D), jnp.float32),
    mesh=_mesh,
    scratch_types=(
        [pltpu.VMEM((_NCH, _C), jnp.int32)]
        + [pltpu.VMEM((_C,), jnp.int32)] * 3
        + [pltpu.VMEM((_C, _D), jnp.float32)] * 3
        + [pltpu.VMEM_SHARED((_N, _D), jnp.float32)]
        + [pltpu.SemaphoreType.DMA] * 6
    ),
    compiler_params=_sc_params,
)


def _prep_body(deg_ref, x_ref, dis_ref, xn_ref):
    deg = deg_ref[0] + deg_ref[1] + 1.0
    dis = lax.rsqrt(deg)
    dis_ref[...] = dis
    xn_ref[...] = x_ref[...] * dis


_prep_call = pl.pallas_call(
    _prep_body,
    grid=(_N // _R,),
    in_specs=[
        pl.BlockSpec((_NC, _R, 1), lambda i: (0, i, 0)),
        pl.BlockSpec((_R, _D), lambda i: (i, 0)),
    ],
    out_specs=[
        pl.BlockSpec((_R, 1), lambda i: (i, 0)),
        pl.BlockSpec((_R, _D), lambda i: (i, 0)),
    ],
    out_shape=[
        jax.ShapeDtypeStruct((_N, 1), jnp.float32),
        jax.ShapeDtypeStruct((_N, _D), jnp.float32),
    ],
)


def _ct(lhs, rhs):
    return lax.dot_general(lhs, rhs, (((1,), (1,)), ((), ())),
                           preferred_element_type=jnp.float32)


def _gated_experts(dis_ref, yn_ref, s_ref, tf_ref, G_ref, W_ref, b_ref):
    dis = dis_ref[...]
    agg = dis * (s_ref[0] + s_ref[1] - yn_ref[...])
    logits = _ct(tf_ref[...], G_ref[...]) * (1.0 / _TEMP)
    m = jnp.max(logits, axis=-1, keepdims=True)
    e = jnp.exp(logits - m)
    gate = e / jnp.sum(e, axis=-1, keepdims=True)
    h = jnp.zeros_like(agg)
    for i in range(3):
        eo = jnp.maximum(_ct(agg, W_ref[i]) + b_ref[i], 0.0)
        h = h + gate[:, i:i + 1] * eo
    return h, dis


def _layer1_body(dis_ref, yn_ref, s_ref, tf_ref, G_ref, W_ref, b_ref, out_ref):
    h, dis = _gated_experts(dis_ref, yn_ref, s_ref, tf_ref, G_ref, W_ref, b_ref)
    out_ref[...] = h * dis


def _layer2_body(dis_ref, yn_ref, s_ref, tf_ref, G_ref, W_ref, b_ref,
                 fcW_ref, fcb_ref, out_ref):
    h, _ = _gated_experts(dis_ref, yn_ref, s_ref, tf_ref, G_ref, W_ref, b_ref)
    out_ref[...] = _ct(h, fcW_ref[...]) + fcb_ref[...]


_layer_in_specs = [
    pl.BlockSpec((_R, 1), lambda i: (i, 0)),
    pl.BlockSpec((_R, _D), lambda i: (i, 0)),
    pl.BlockSpec((_NC, _R, _D), lambda i: (0, i, 0)),
    pl.BlockSpec((_R, 4), lambda i: (i, 0)),
    pl.BlockSpec((3, 4), lambda i: (0, 0)),
    pl.BlockSpec((3, _D, _D), lambda i: (0, 0, 0)),
    pl.BlockSpec((3, _D), lambda i: (0, 0)),
]

_layer1_call = pl.pallas_call(
    _layer1_body,
    grid=(_N // _R,),
    in_specs=_layer_in_specs,
    out_specs=pl.BlockSpec((_R, _D), lambda i: (i, 0)),
    out_shape=jax.ShapeDtypeStruct((_N, _D), jnp.float32),
)

_layer2_call = pl.pallas_call(
    _layer2_body,
    grid=(_N // _R,),
    in_specs=_layer_in_specs + [
        pl.BlockSpec((_D, _D), lambda i: (0, 0)),
        pl.BlockSpec((1, _D), lambda i: (0, 0)),
    ],
    out_specs=pl.BlockSpec((_R, _D), lambda i: (i, 0)),
    out_shape=jax.ShapeDtypeStruct((_N, _D), jnp.float32),
)


def kernel(x, edge_index, top_features, W1, b1, W2, b2, G1, G2, fcW, fcb):
    src = edge_index[0]
    dst = edge_index[1]
    src3 = src.reshape(_NW, _NCH, _C)

    deg2 = _deg_call(dst).reshape(_NC, _NPAD)[:, :_N].reshape(_NC, _N, 1)
    dis, xn = _prep_call(deg2, x)

    s1 = _scat_call(xn, src3, dst)
    hn = _layer1_call(dis, xn, s1, top_features, G1, W1, b1)

    s2 = _scat_call(hn, src3, dst)
    return _layer2_call(dis, hn, s2, top_features, G2, W2, b2,
                        fcW, fcb.reshape(1, _D))
